# baseline (speedup 1.0000x reference)
import functools

import jax
import jax.numpy as jnp
from jax import lax
from jax.experimental import pallas as pl
from jax.experimental.pallas import tpu as pltpu
from jax.experimental.pallas import tpu_sc as plsc

N_NODES = 10000
NP = 10240
E = 320000
EP = 327680
EPT = EP // 32
ECH = 64
NCH = EPT // ECH
RBUF = 5
EBUF = 10
NCHT = 2 * NCH
GB = 64
D0 = 128
D1 = 256
RPT = NP // 32
SROWS = NP // 16

def _pass1_body(table_h, ids_h, edges_h, x_h, deg_h,
                gidx_v, grows_v, cidx0, cidx1, ones_v, zeros_v, deg_sh,
                sem, isem0, isem1, dsem0, dsem1):
    c = lax.axis_index("c")
    s = lax.axis_index("s")
    wid = c * 16 + s

    def zfill(i, _):
        zeros_v[pl.ds(i * 16, 16)] = jnp.zeros((16,), jnp.float32)
        return 0
    lax.fori_loop(0, SROWS // 16, zfill, 0)
    for i in range(ECH // 16):
        ones_v[pl.ds(i * 16, 16)] = jnp.ones((16,), jnp.float32)

    pltpu.sync_copy(zeros_v, deg_sh.at[pl.ds(s * SROWS, SROWS)])
    plsc.subcore_barrier()

    nb = wid * RPT

    def gbody(i, _):
        off = nb + i * 80
        pltpu.sync_copy(ids_h.at[pl.ds(off, 80)], gidx_v)
        pltpu.async_copy(table_h.at[gidx_v], grows_v, sem).wait()
        pltpu.sync_copy(grows_v, x_h.at[pl.ds(off, 80)])
        return 0
    lax.fori_loop(0, RPT // 80, gbody, 0)

    cb = wid * NCH
    cidx = (cidx0, cidx1)
    isem = (isem0, isem1)
    dsem = (dsem0, dsem1)
    pltpu.async_copy(edges_h.at[cb, 1], cidx0, isem0)

    @pl.loop(0, NCH, step=2)
    def _(j):
        for b in range(2):
            cur = j + b
            nxt = 1 - b

            @pl.when(cur >= 1)
            def _():
                pltpu.make_async_copy(ones_v, deg_sh.at[cidx[nxt]],
                                      dsem[nxt]).wait()

            @pl.when(cur + 1 < NCH)
            def _():
                pltpu.async_copy(edges_h.at[cb + cur + 1, 1], cidx[nxt],
                                 isem[nxt])

            pltpu.make_async_copy(edges_h.at[cb + cur, 1], cidx[b],
                                  isem[b]).wait()
            pltpu.async_copy(ones_v, deg_sh.at[cidx[b]], dsem[b], add=True)

    pltpu.make_async_copy(ones_v, deg_sh.at[cidx[1]], dsem[1]).wait()
    plsc.subcore_barrier()
    pltpu.sync_copy(deg_sh.at[pl.ds(s * SROWS, SROWS)],
                    deg_h.at[pl.ds(c * NP + s * SROWS, SROWS)])


def _prop_body(y_h, edges_h, z_h, eidx, rows, z_sh, isem, gsem, ssem):
    c = lax.axis_index("c")
    s = lax.axis_index("s")

    @pl.when(c == 0)
    def _():
        def zb(i, _):
            for jj in range(D0 // 16):
                rows[0][i, pl.ds(jj * 16, 16)] = jnp.zeros((16,),
                                                           jnp.float32)
            return 0
        lax.fori_loop(0, ECH, zb, 0)

        def zs(k, _):
            pltpu.sync_copy(rows[0],
                            z_sh.at[pl.ds(s * SROWS + k * ECH, ECH)])
            return 0
        lax.fori_loop(0, SROWS // ECH, zs, 0)
        plsc.subcore_barrier()

        cb = s * NCHT

        def drain_scatter(rb, ib):
            pltpu.make_async_copy(rows[rb], z_sh.at[eidx[ib].at[1]],
                                  ssem[rb]).wait()

        for k in range(4):
            pltpu.async_copy(edges_h.at[cb + k], eidx[k], isem[k])
        for k in range(3):
            pltpu.make_async_copy(edges_h.at[cb + k], eidx[k],
                                  isem[k]).wait()
            pltpu.async_copy(y_h.at[eidx[k].at[0]], rows[k], gsem[k])

        def blk(i, _):
            j = i * EBUF
            for b in range(EBUF):
                cur = j + b

                @pl.when(cur >= 2)
                def _():
                    drain_scatter((b - 2) % RBUF, (b - 2) % EBUF)

                @pl.when(cur + 4 < NCHT)
                def _():
                    pltpu.async_copy(edges_h.at[cb + cur + 4],
                                     eidx[(b + 4) % EBUF],
                                     isem[(b + 4) % EBUF])

                @pl.when(cur + 3 < NCHT)
                def _():
                    pltpu.make_async_copy(edges_h.at[cb + cur + 3],
                                          eidx[(b + 3) % EBUF],
                                          isem[(b + 3) % EBUF]).wait()
                    pltpu.async_copy(y_h.at[eidx[(b + 3) % EBUF].at[0]],
                                     rows[(b + 3) % RBUF],
                                     gsem[(b + 3) % RBUF])

                pltpu.make_async_copy(y_h.at[eidx[b % EBUF].at[0]],
                                      rows[b % RBUF], gsem[b % RBUF]).wait()
                pltpu.async_copy(rows[b % RBUF],
                                 z_sh.at[eidx[b % EBUF].at[1]],
                                 ssem[b % RBUF], add=True)
            return 0
        lax.fori_loop(0, NCHT // EBUF, blk, 0)

        drain_scatter((NCHT - 2) % RBUF, (NCHT - 2) % EBUF)
        drain_scatter((NCHT - 1) % RBUF, (NCHT - 1) % EBUF)
        plsc.subcore_barrier()

        def wb(k, _):
            pltpu.sync_copy(z_sh.at[pl.ds(s * SROWS + k * ECH, ECH)],
                            z_h.at[pl.ds(s * SROWS + k * ECH, ECH)])
            return 0
        lax.fori_loop(0, SROWS // ECH, wb, 0)


@functools.cache
def _sc_pass1():
    mesh = plsc.VectorSubcoreMesh(core_axis_name="c", subcore_axis_name="s")
    return pl.kernel(
        _pass1_body,
        out_type=[jax.ShapeDtypeStruct((NP, D0), jnp.float32),
                  jax.ShapeDtypeStruct((2 * NP,), jnp.float32)],
        mesh=mesh,
        scratch_types=[
            pltpu.VMEM((80,), jnp.int32),
            pltpu.VMEM((80, D0), jnp.float32),
            pltpu.VMEM((ECH,), jnp.int32),
            pltpu.VMEM((ECH,), jnp.int32),
            pltpu.VMEM((ECH,), jnp.float32),
            pltpu.VMEM((SROWS,), jnp.float32),
            pltpu.VMEM_SHARED((NP,), jnp.float32),
            pltpu.SemaphoreType.DMA,
            pltpu.SemaphoreType.DMA,
            pltpu.SemaphoreType.DMA,
            pltpu.SemaphoreType.DMA,
            pltpu.SemaphoreType.DMA,
        ],
    )


@functools.cache
def _sc_prop():
    mesh = plsc.VectorSubcoreMesh(core_axis_name="c", subcore_axis_name="s")
    return pl.kernel(
        _prop_body,
        out_type=jax.ShapeDtypeStruct((NP, D0), jnp.float32),
        mesh=mesh,
        scratch_types=[
            tuple(pltpu.VMEM((2, ECH), jnp.int32) for _ in range(EBUF)),
            tuple(pltpu.VMEM((ECH, D0), jnp.float32) for _ in range(RBUF)),
            pltpu.VMEM_SHARED((NP, D0), jnp.float32),
            tuple(pltpu.SemaphoreType.DMA for _ in range(EBUF)),
            tuple(pltpu.SemaphoreType.DMA for _ in range(RBUF)),
            tuple(pltpu.SemaphoreType.DMA for _ in range(RBUF)),
        ],
    )


ROWS_BLK = 1024
GRID = NP // ROWS_BLK


def _t1_body(x_ref, degr_ref, y1_ref, d_ref):
    deg = jnp.sum(degr_ref[...], axis=1, keepdims=True) + 1.0
    d = lax.rsqrt(deg)
    d_ref[...] = d
    y1_ref[...] = x_ref[...] * d


def _t2_body(za, y1, d, wi, bi, wo, y2):
    p = (za[...] + y1[...]) * d[...]
    h = jnp.dot(p, wi[...], preferred_element_type=jnp.float32) + bi[...]
    h = jnp.maximum(h, 0.0)
    t = jnp.dot(h, wo[...], preferred_element_type=jnp.float32)
    y2[...] = t * d[...]


def _t3_body(za, y2, d, bo, batch, out_ref, sums, counts):
    i = pl.program_id(0)
    node = (za[...] + y2[...]) * d[...] + bo[...]
    oh = (batch[...] == lax.broadcasted_iota(jnp.int32, (ROWS_BLK, GB), 1))
    oh = oh.astype(jnp.float32)
    dn = (((0,), (0,)), ((), ()))
    ps = lax.dot_general(oh, node, dn, preferred_element_type=jnp.float32)
    pc = lax.dot_general(oh, jnp.ones((ROWS_BLK, D0), jnp.float32), dn,
                         preferred_element_type=jnp.float32)

    @pl.when(i == 0)
    def _():
        sums[...] = jnp.zeros_like(sums)
        counts[...] = jnp.zeros_like(counts)

    sums[...] += ps
    counts[...] += pc

    @pl.when(i == GRID - 1)
    def _():
        out_ref[...] = sums[...] / jnp.maximum(counts[...], 1.0)


def _row_spec(cols):
    return pl.BlockSpec((ROWS_BLK, cols), lambda i: (i, 0))


def _full_spec(r, c):
    return pl.BlockSpec((r, c), lambda i: (0, 0))


_t1 = pl.pallas_call(
    _t1_body,
    grid=(GRID,),
    in_specs=[_row_spec(D0), _row_spec(2)],
    out_specs=[_row_spec(D0), _row_spec(1)],
    out_shape=[jax.ShapeDtypeStruct((NP, D0), jnp.float32),
               jax.ShapeDtypeStruct((NP, 1), jnp.float32)],
)

_t2 = pl.pallas_call(
    _t2_body,
    grid=(GRID,),
    in_specs=[_row_spec(D0), _row_spec(D0), _row_spec(1),
              _full_spec(D0, D1), _full_spec(1, D1), _full_spec(D1, D0)],
    out_specs=_row_spec(D0),
    out_shape=jax.ShapeDtypeStruct((NP, D0), jnp.float32),
)

_t3 = pl.pallas_call(
    _t3_body,
    grid=(GRID,),
    in_specs=[_row_spec(D0), _row_spec(D0), _row_spec(1),
              _full_spec(1, D0), _row_spec(1)],
    out_specs=_full_spec(GB, D0),
    out_shape=jax.ShapeDtypeStruct((GB, D0), jnp.float32),
    scratch_shapes=[pltpu.VMEM((GB, D0), jnp.float32),
                    pltpu.VMEM((GB, D0), jnp.float32)],
)


@jax.jit
def kernel(node_ids, edge_index, batch, embed_table, W_in, b_in, W_out, b_out):
    ids_p = jnp.pad(node_ids, (0, NP - N_NODES))
    row_p = jnp.pad(edge_index[0], (0, EP - E))
    pad_cols = N_NODES + (jnp.arange(EP - E, dtype=jnp.int32) % (NP - N_NODES))
    col_p = jnp.concatenate([edge_index[1], pad_cols])
    batch_p = jnp.pad(batch, (0, NP - N_NODES), constant_values=GB)
    edges_c = (jnp.stack([row_p, col_p])
               .reshape(2, EP // ECH, ECH).transpose(1, 0, 2))

    x, deg_flat = _sc_pass1()(embed_table, ids_p, edges_c)
    deg_r = deg_flat.reshape(2, NP).T

    y1, d = _t1(x, deg_r)
    z1 = _sc_prop()(y1, edges_c)
    y2 = _t2(z1, y1, d, W_in, b_in.reshape(1, D1), W_out)
    z2 = _sc_prop()(y2, edges_c)
    out = _t3(z2, y2, d, b_out.reshape(1, D0), batch_p.reshape(NP, 1))
    return out


# speedup vs baseline: 1.1918x; 1.1918x over previous
import functools

import jax
import jax.numpy as jnp
from jax import lax
from jax.experimental import pallas as pl
from jax.experimental.pallas import tpu as pltpu
from jax.experimental.pallas import tpu_sc as plsc

N_NODES = 10000
NP = 10240
E = 320000
EP = 327680
EPT = EP // 32
ECH = 64
NCH = EPT // ECH
RBUF = 5
EBUF = 10
NCH0 = 280
NCH1 = 2 * NCH - NCH0
GB = 64
D0 = 128
D1 = 256
RPT = NP // 32
SROWS = NP // 16

def _pass1_body(table_h, ids_h, edges_h, x_h, deg_h,
                gidx_v, grows_v, cidx0, cidx1, ones_v, zeros_v, deg_sh,
                sem, isem0, isem1, dsem0, dsem1):
    c = lax.axis_index("c")
    s = lax.axis_index("s")
    wid = c * 16 + s

    def zfill(i, _):
        zeros_v[pl.ds(i * 16, 16)] = jnp.zeros((16,), jnp.float32)
        return 0
    lax.fori_loop(0, SROWS // 16, zfill, 0)
    for i in range(ECH // 16):
        ones_v[pl.ds(i * 16, 16)] = jnp.ones((16,), jnp.float32)

    pltpu.sync_copy(zeros_v, deg_sh.at[pl.ds(s * SROWS, SROWS)])
    plsc.subcore_barrier()

    nb = wid * RPT

    def gbody(i, _):
        off = nb + i * 80
        pltpu.sync_copy(ids_h.at[pl.ds(off, 80)], gidx_v)
        pltpu.async_copy(table_h.at[gidx_v], grows_v, sem).wait()
        pltpu.sync_copy(grows_v, x_h.at[pl.ds(off, 80)])
        return 0
    lax.fori_loop(0, RPT // 80, gbody, 0)

    cb = wid * NCH
    cidx = (cidx0, cidx1)
    isem = (isem0, isem1)
    dsem = (dsem0, dsem1)
    pltpu.async_copy(edges_h.at[cb, 1], cidx0, isem0)

    @pl.loop(0, NCH, step=2)
    def _(j):
        for b in range(2):
            cur = j + b
            nxt = 1 - b

            @pl.when(cur >= 1)
            def _():
                pltpu.make_async_copy(ones_v, deg_sh.at[cidx[nxt]],
                                      dsem[nxt]).wait()

            @pl.when(cur + 1 < NCH)
            def _():
                pltpu.async_copy(edges_h.at[cb + cur + 1, 1], cidx[nxt],
                                 isem[nxt])

            pltpu.make_async_copy(edges_h.at[cb + cur, 1], cidx[b],
                                  isem[b]).wait()
            pltpu.async_copy(ones_v, deg_sh.at[cidx[b]], dsem[b], add=True)

    pltpu.make_async_copy(ones_v, deg_sh.at[cidx[1]], dsem[1]).wait()
    plsc.subcore_barrier()
    pltpu.sync_copy(deg_sh.at[pl.ds(s * SROWS, SROWS)],
                    deg_h.at[pl.ds(c * NP + s * SROWS, SROWS)])


def _prop_body(y_h, edges_h, z_h, eidx, rows, z_sh, isem, gsem, ssem):
    c = lax.axis_index("c")
    s = lax.axis_index("s")

    def zb(i, _):
        for jj in range(D0 // 16):
            rows[0][i, pl.ds(jj * 16, 16)] = jnp.zeros((16,), jnp.float32)
        return 0
    lax.fori_loop(0, ECH, zb, 0)

    def zs(k, _):
        pltpu.sync_copy(rows[0], z_sh.at[pl.ds(s * SROWS + k * ECH, ECH)])
        return 0
    lax.fori_loop(0, SROWS // ECH, zs, 0)
    plsc.subcore_barrier()

    nch = jnp.where(c == 0, NCH0, NCH1)
    cb = jnp.where(c == 0, s * NCH0, 16 * NCH0 + s * NCH1)

    def drain_scatter(rb, ib):
        pltpu.make_async_copy(rows[rb], z_sh.at[eidx[ib].at[1]],
                              ssem[rb]).wait()

    for k in range(4):
        pltpu.async_copy(edges_h.at[cb + k], eidx[k], isem[k])
    for k in range(3):
        pltpu.make_async_copy(edges_h.at[cb + k], eidx[k], isem[k]).wait()
        pltpu.async_copy(y_h.at[eidx[k].at[0]], rows[k], gsem[k])

    def blk(i, _):
        j = i * EBUF
        for b in range(EBUF):
            cur = j + b

            @pl.when(cur >= 2)
            def _():
                drain_scatter((b - 2) % RBUF, (b - 2) % EBUF)

            @pl.when(cur + 4 < nch)
            def _():
                pltpu.async_copy(edges_h.at[cb + cur + 4],
                                 eidx[(b + 4) % EBUF], isem[(b + 4) % EBUF])

            @pl.when(cur + 3 < nch)
            def _():
                pltpu.make_async_copy(edges_h.at[cb + cur + 3],
                                      eidx[(b + 3) % EBUF],
                                      isem[(b + 3) % EBUF]).wait()
                pltpu.async_copy(y_h.at[eidx[(b + 3) % EBUF].at[0]],
                                 rows[(b + 3) % RBUF], gsem[(b + 3) % RBUF])

            pltpu.make_async_copy(y_h.at[eidx[b % EBUF].at[0]],
                                  rows[b % RBUF], gsem[b % RBUF]).wait()
            pltpu.async_copy(rows[b % RBUF], z_sh.at[eidx[b % EBUF].at[1]],
                             ssem[b % RBUF], add=True)
        return 0
    lax.fori_loop(0, nch // EBUF, blk, 0)

    drain_scatter((NCH0 - 2) % RBUF, (NCH0 - 2) % EBUF)
    drain_scatter((NCH0 - 1) % RBUF, (NCH0 - 1) % EBUF)
    plsc.subcore_barrier()

    def wb(k, _):
        pltpu.sync_copy(z_sh.at[pl.ds(s * SROWS + k * ECH, ECH)],
                        z_h.at[pl.ds(c * NP + s * SROWS + k * ECH, ECH)])
        return 0
    lax.fori_loop(0, SROWS // ECH, wb, 0)


@functools.cache
def _sc_pass1():
    mesh = plsc.VectorSubcoreMesh(core_axis_name="c", subcore_axis_name="s")
    return pl.kernel(
        _pass1_body,
        out_type=[jax.ShapeDtypeStruct((NP, D0), jnp.float32),
                  jax.ShapeDtypeStruct((2 * NP,), jnp.float32)],
        mesh=mesh,
        scratch_types=[
            pltpu.VMEM((80,), jnp.int32),
            pltpu.VMEM((80, D0), jnp.float32),
            pltpu.VMEM((ECH,), jnp.int32),
            pltpu.VMEM((ECH,), jnp.int32),
            pltpu.VMEM((ECH,), jnp.float32),
            pltpu.VMEM((SROWS,), jnp.float32),
            pltpu.VMEM_SHARED((NP,), jnp.float32),
            pltpu.SemaphoreType.DMA,
            pltpu.SemaphoreType.DMA,
            pltpu.SemaphoreType.DMA,
            pltpu.SemaphoreType.DMA,
            pltpu.SemaphoreType.DMA,
        ],
    )


@functools.cache
def _sc_prop():
    mesh = plsc.VectorSubcoreMesh(core_axis_name="c", subcore_axis_name="s")
    return pl.kernel(
        _prop_body,
        out_type=jax.ShapeDtypeStruct((2 * NP, D0), jnp.float32),
        mesh=mesh,
        scratch_types=[
            tuple(pltpu.VMEM((2, ECH), jnp.int32) for _ in range(EBUF)),
            tuple(pltpu.VMEM((ECH, D0), jnp.float32) for _ in range(RBUF)),
            pltpu.VMEM_SHARED((NP, D0), jnp.float32),
            tuple(pltpu.SemaphoreType.DMA for _ in range(EBUF)),
            tuple(pltpu.SemaphoreType.DMA for _ in range(RBUF)),
            tuple(pltpu.SemaphoreType.DMA for _ in range(RBUF)),
        ],
    )


ROWS_BLK = 1024
GRID = NP // ROWS_BLK


def _t1_body(x_ref, degr_ref, y1_ref, d_ref):
    deg = jnp.sum(degr_ref[...], axis=1, keepdims=True) + 1.0
    d = lax.rsqrt(deg)
    d_ref[...] = d
    y1_ref[...] = x_ref[...] * d


def _t2_body(za, zb, y1, d, wi, bi, wo, y2):
    p = (za[...] + zb[...] + y1[...]) * d[...]
    h = jnp.dot(p, wi[...], preferred_element_type=jnp.float32) + bi[...]
    h = jnp.maximum(h, 0.0)
    t = jnp.dot(h, wo[...], preferred_element_type=jnp.float32)
    y2[...] = t * d[...]


def _t3_body(za, zb, y2, d, bo, batch, out_ref, sums, counts):
    i = pl.program_id(0)
    node = (za[...] + zb[...] + y2[...]) * d[...] + bo[...]
    oh = (batch[...] == lax.broadcasted_iota(jnp.int32, (ROWS_BLK, GB), 1))
    oh = oh.astype(jnp.float32)
    dn = (((0,), (0,)), ((), ()))
    ps = lax.dot_general(oh, node, dn, preferred_element_type=jnp.float32)
    pc = lax.dot_general(oh, jnp.ones((ROWS_BLK, D0), jnp.float32), dn,
                         preferred_element_type=jnp.float32)

    @pl.when(i == 0)
    def _():
        sums[...] = jnp.zeros_like(sums)
        counts[...] = jnp.zeros_like(counts)

    sums[...] += ps
    counts[...] += pc

    @pl.when(i == GRID - 1)
    def _():
        out_ref[...] = sums[...] / jnp.maximum(counts[...], 1.0)


def _row_spec(cols):
    return pl.BlockSpec((ROWS_BLK, cols), lambda i: (i, 0))


def _full_spec(r, c):
    return pl.BlockSpec((r, c), lambda i: (0, 0))


_t1 = pl.pallas_call(
    _t1_body,
    grid=(GRID,),
    in_specs=[_row_spec(D0), _row_spec(2)],
    out_specs=[_row_spec(D0), _row_spec(1)],
    out_shape=[jax.ShapeDtypeStruct((NP, D0), jnp.float32),
               jax.ShapeDtypeStruct((NP, 1), jnp.float32)],
)

_t2 = pl.pallas_call(
    _t2_body,
    grid=(GRID,),
    in_specs=[_row_spec(D0), _row_spec(D0), _row_spec(D0), _row_spec(1),
              _full_spec(D0, D1), _full_spec(1, D1), _full_spec(D1, D0)],
    out_specs=_row_spec(D0),
    out_shape=jax.ShapeDtypeStruct((NP, D0), jnp.float32),
)

_t3 = pl.pallas_call(
    _t3_body,
    grid=(GRID,),
    in_specs=[_row_spec(D0), _row_spec(D0), _row_spec(D0), _row_spec(1),
              _full_spec(1, D0), _row_spec(1)],
    out_specs=_full_spec(GB, D0),
    out_shape=jax.ShapeDtypeStruct((GB, D0), jnp.float32),
    scratch_shapes=[pltpu.VMEM((GB, D0), jnp.float32),
                    pltpu.VMEM((GB, D0), jnp.float32)],
)


@jax.jit
def kernel(node_ids, edge_index, batch, embed_table, W_in, b_in, W_out, b_out):
    ids_p = jnp.pad(node_ids, (0, NP - N_NODES))
    row_p = jnp.pad(edge_index[0], (0, EP - E))
    pad_cols = N_NODES + (jnp.arange(EP - E, dtype=jnp.int32) % (NP - N_NODES))
    col_p = jnp.concatenate([edge_index[1], pad_cols])
    batch_p = jnp.pad(batch, (0, NP - N_NODES), constant_values=GB)
    edges_c = (jnp.stack([row_p, col_p])
               .reshape(2, EP // ECH, ECH).transpose(1, 0, 2))

    x, deg_flat = _sc_pass1()(embed_table, ids_p, edges_c)
    deg_r = deg_flat.reshape(2, NP).T

    y1, d = _t1(x, deg_r)
    z1 = _sc_prop()(y1, edges_c)
    y2 = _t2(z1[:NP], z1[NP:], y1, d, W_in, b_in.reshape(1, D1), W_out)
    z2 = _sc_prop()(y2, edges_c)
    out = _t3(z2[:NP], z2[NP:], y2, d, b_out.reshape(1, D0),
              batch_p.reshape(NP, 1))
    return out


# speedup vs baseline: 1.2265x; 1.0291x over previous
import functools

import jax
import jax.numpy as jnp
from jax import lax
from jax.experimental import pallas as pl
from jax.experimental.pallas import tpu as pltpu
from jax.experimental.pallas import tpu_sc as plsc

N_NODES = 10000
NP = 10240
E = 320000
EP = 327680
EPT = EP // 32
ECH = 64
NCH = EPT // ECH
RBUF = 5
EBUF = 10
NCH0 = 300
NCH1 = 2 * NCH - NCH0
GB = 64
D0 = 128
D1 = 256
RPT = NP // 32
SROWS = NP // 16

def _pass1_body(table_h, ids_h, edges_h, x_h, deg_h,
                gidx_v, grows_v, cidx0, cidx1, ones_v, zeros_v, deg_sh,
                sem, isem0, isem1, dsem0, dsem1):
    c = lax.axis_index("c")
    s = lax.axis_index("s")
    wid = c * 16 + s

    def zfill(i, _):
        zeros_v[pl.ds(i * 16, 16)] = jnp.zeros((16,), jnp.float32)
        return 0
    lax.fori_loop(0, SROWS // 16, zfill, 0)
    for i in range(ECH // 16):
        ones_v[pl.ds(i * 16, 16)] = jnp.ones((16,), jnp.float32)

    pltpu.sync_copy(zeros_v, deg_sh.at[pl.ds(s * SROWS, SROWS)])
    plsc.subcore_barrier()

    nb = wid * RPT

    def gbody(i, _):
        off = nb + i * 80
        pltpu.sync_copy(ids_h.at[pl.ds(off, 80)], gidx_v)
        pltpu.async_copy(table_h.at[gidx_v], grows_v, sem).wait()
        pltpu.sync_copy(grows_v, x_h.at[pl.ds(off, 80)])
        return 0
    lax.fori_loop(0, RPT // 80, gbody, 0)

    cb = wid * NCH
    cidx = (cidx0, cidx1)
    isem = (isem0, isem1)
    dsem = (dsem0, dsem1)
    pltpu.async_copy(edges_h.at[cb, 1], cidx0, isem0)

    @pl.loop(0, NCH, step=2)
    def _(j):
        for b in range(2):
            cur = j + b
            nxt = 1 - b

            @pl.when(cur >= 1)
            def _():
                pltpu.make_async_copy(ones_v, deg_sh.at[cidx[nxt]],
                                      dsem[nxt]).wait()

            @pl.when(cur + 1 < NCH)
            def _():
                pltpu.async_copy(edges_h.at[cb + cur + 1, 1], cidx[nxt],
                                 isem[nxt])

            pltpu.make_async_copy(edges_h.at[cb + cur, 1], cidx[b],
                                  isem[b]).wait()
            pltpu.async_copy(ones_v, deg_sh.at[cidx[b]], dsem[b], add=True)

    pltpu.make_async_copy(ones_v, deg_sh.at[cidx[1]], dsem[1]).wait()
    plsc.subcore_barrier()
    pltpu.sync_copy(deg_sh.at[pl.ds(s * SROWS, SROWS)],
                    deg_h.at[pl.ds(c * NP + s * SROWS, SROWS)])


def _prop_body(y_h, edges_h, z_h, eidx, rows, z_sh, isem, gsem, ssem):
    c = lax.axis_index("c")
    s = lax.axis_index("s")

    def zb(i, _):
        for jj in range(D0 // 16):
            rows[0][i, pl.ds(jj * 16, 16)] = jnp.zeros((16,), jnp.float32)
        return 0
    lax.fori_loop(0, ECH, zb, 0)

    def zs(k, _):
        pltpu.sync_copy(rows[0], z_sh.at[pl.ds(s * SROWS + k * ECH, ECH)])
        return 0
    lax.fori_loop(0, SROWS // ECH, zs, 0)
    plsc.subcore_barrier()

    nch = jnp.where(c == 0, NCH0, NCH1)
    cb = jnp.where(c == 0, s * NCH0, 16 * NCH0 + s * NCH1)

    def drain_scatter(rb, ib):
        pltpu.make_async_copy(rows[rb], z_sh.at[eidx[ib].at[1]],
                              ssem[rb]).wait()

    for k in range(4):
        pltpu.async_copy(edges_h.at[cb + k], eidx[k], isem[k])
    for k in range(3):
        pltpu.make_async_copy(edges_h.at[cb + k], eidx[k], isem[k]).wait()
        pltpu.async_copy(y_h.at[eidx[k].at[0]], rows[k], gsem[k])

    def blk(i, _):
        j = i * EBUF
        for b in range(EBUF):
            cur = j + b

            @pl.when(cur >= 2)
            def _():
                drain_scatter((b - 2) % RBUF, (b - 2) % EBUF)

            @pl.when(cur + 4 < nch)
            def _():
                pltpu.async_copy(edges_h.at[cb + cur + 4],
                                 eidx[(b + 4) % EBUF], isem[(b + 4) % EBUF])

            @pl.when(cur + 3 < nch)
            def _():
                pltpu.make_async_copy(edges_h.at[cb + cur + 3],
                                      eidx[(b + 3) % EBUF],
                                      isem[(b + 3) % EBUF]).wait()
                pltpu.async_copy(y_h.at[eidx[(b + 3) % EBUF].at[0]],
                                 rows[(b + 3) % RBUF], gsem[(b + 3) % RBUF])

            pltpu.make_async_copy(y_h.at[eidx[b % EBUF].at[0]],
                                  rows[b % RBUF], gsem[b % RBUF]).wait()
            pltpu.async_copy(rows[b % RBUF], z_sh.at[eidx[b % EBUF].at[1]],
                             ssem[b % RBUF], add=True)
        return 0
    lax.fori_loop(0, nch // EBUF, blk, 0)

    drain_scatter((NCH0 - 2) % RBUF, (NCH0 - 2) % EBUF)
    drain_scatter((NCH0 - 1) % RBUF, (NCH0 - 1) % EBUF)
    plsc.subcore_barrier()

    def wb(k, _):
        pltpu.sync_copy(z_sh.at[pl.ds(s * SROWS + k * ECH, ECH)],
                        z_h.at[pl.ds(c * NP + s * SROWS + k * ECH, ECH)])
        return 0
    lax.fori_loop(0, SROWS // ECH, wb, 0)


@functools.cache
def _sc_pass1():
    mesh = plsc.VectorSubcoreMesh(core_axis_name="c", subcore_axis_name="s")
    return pl.kernel(
        _pass1_body,
        out_type=[jax.ShapeDtypeStruct((NP, D0), jnp.float32),
                  jax.ShapeDtypeStruct((2 * NP,), jnp.float32)],
        mesh=mesh,
        scratch_types=[
            pltpu.VMEM((80,), jnp.int32),
            pltpu.VMEM((80, D0), jnp.float32),
            pltpu.VMEM((ECH,), jnp.int32),
            pltpu.VMEM((ECH,), jnp.int32),
            pltpu.VMEM((ECH,), jnp.float32),
            pltpu.VMEM((SROWS,), jnp.float32),
            pltpu.VMEM_SHARED((NP,), jnp.float32),
            pltpu.SemaphoreType.DMA,
            pltpu.SemaphoreType.DMA,
            pltpu.SemaphoreType.DMA,
            pltpu.SemaphoreType.DMA,
            pltpu.SemaphoreType.DMA,
        ],
    )


@functools.cache
def _sc_prop():
    mesh = plsc.VectorSubcoreMesh(core_axis_name="c", subcore_axis_name="s")
    return pl.kernel(
        _prop_body,
        out_type=jax.ShapeDtypeStruct((2 * NP, D0), jnp.float32),
        mesh=mesh,
        scratch_types=[
            tuple(pltpu.VMEM((2, ECH), jnp.int32) for _ in range(EBUF)),
            tuple(pltpu.VMEM((ECH, D0), jnp.float32) for _ in range(RBUF)),
            pltpu.VMEM_SHARED((NP, D0), jnp.float32),
            tuple(pltpu.SemaphoreType.DMA for _ in range(EBUF)),
            tuple(pltpu.SemaphoreType.DMA for _ in range(RBUF)),
            tuple(pltpu.SemaphoreType.DMA for _ in range(RBUF)),
        ],
    )


ROWS_BLK = 1024
GRID = NP // ROWS_BLK


def _t1_body(x_ref, degr_ref, y1_ref, d_ref):
    deg = jnp.sum(degr_ref[...], axis=1, keepdims=True) + 1.0
    d = lax.rsqrt(deg)
    d_ref[...] = d
    y1_ref[...] = x_ref[...] * d


def _t2_body(za, zb, y1, d, wi, bi, wo, y2):
    p = (za[...] + zb[...] + y1[...]) * d[...]
    h = jnp.dot(p, wi[...], preferred_element_type=jnp.float32) + bi[...]
    h = jnp.maximum(h, 0.0)
    t = jnp.dot(h, wo[...], preferred_element_type=jnp.float32)
    y2[...] = t * d[...]


def _t3_body(za, zb, y2, d, bo, batch, out_ref, sums, counts):
    i = pl.program_id(0)
    node = (za[...] + zb[...] + y2[...]) * d[...] + bo[...]
    oh = (batch[...] == lax.broadcasted_iota(jnp.int32, (ROWS_BLK, GB), 1))
    oh = oh.astype(jnp.float32)
    dn = (((0,), (0,)), ((), ()))
    ps = lax.dot_general(oh, node, dn, preferred_element_type=jnp.float32)
    pc = lax.dot_general(oh, jnp.ones((ROWS_BLK, D0), jnp.float32), dn,
                         preferred_element_type=jnp.float32)

    @pl.when(i == 0)
    def _():
        sums[...] = jnp.zeros_like(sums)
        counts[...] = jnp.zeros_like(counts)

    sums[...] += ps
    counts[...] += pc

    @pl.when(i == GRID - 1)
    def _():
        out_ref[...] = sums[...] / jnp.maximum(counts[...], 1.0)


def _row_spec(cols):
    return pl.BlockSpec((ROWS_BLK, cols), lambda i: (i, 0))


def _full_spec(r, c):
    return pl.BlockSpec((r, c), lambda i: (0, 0))


_t1 = pl.pallas_call(
    _t1_body,
    grid=(GRID,),
    in_specs=[_row_spec(D0), _row_spec(2)],
    out_specs=[_row_spec(D0), _row_spec(1)],
    out_shape=[jax.ShapeDtypeStruct((NP, D0), jnp.float32),
               jax.ShapeDtypeStruct((NP, 1), jnp.float32)],
)

_t2 = pl.pallas_call(
    _t2_body,
    grid=(GRID,),
    in_specs=[_row_spec(D0), _row_spec(D0), _row_spec(D0), _row_spec(1),
              _full_spec(D0, D1), _full_spec(1, D1), _full_spec(D1, D0)],
    out_specs=_row_spec(D0),
    out_shape=jax.ShapeDtypeStruct((NP, D0), jnp.float32),
)

_t3 = pl.pallas_call(
    _t3_body,
    grid=(GRID,),
    in_specs=[_row_spec(D0), _row_spec(D0), _row_spec(D0), _row_spec(1),
              _full_spec(1, D0), _row_spec(1)],
    out_specs=_full_spec(GB, D0),
    out_shape=jax.ShapeDtypeStruct((GB, D0), jnp.float32),
    scratch_shapes=[pltpu.VMEM((GB, D0), jnp.float32),
                    pltpu.VMEM((GB, D0), jnp.float32)],
)


@jax.jit
def kernel(node_ids, edge_index, batch, embed_table, W_in, b_in, W_out, b_out):
    ids_p = jnp.pad(node_ids, (0, NP - N_NODES))
    row_p = jnp.pad(edge_index[0], (0, EP - E))
    pad_cols = N_NODES + (jnp.arange(EP - E, dtype=jnp.int32) % (NP - N_NODES))
    col_p = jnp.concatenate([edge_index[1], pad_cols])
    batch_p = jnp.pad(batch, (0, NP - N_NODES), constant_values=GB)
    edges_c = (jnp.stack([row_p, col_p])
               .reshape(2, EP // ECH, ECH).transpose(1, 0, 2))

    x, deg_flat = _sc_pass1()(embed_table, ids_p, edges_c)
    deg_r = deg_flat.reshape(2, NP).T

    y1, d = _t1(x, deg_r)
    z1 = _sc_prop()(y1, edges_c)
    y2 = _t2(z1[:NP], z1[NP:], y1, d, W_in, b_in.reshape(1, D1), W_out)
    z2 = _sc_prop()(y2, edges_c)
    out = _t3(z2[:NP], z2[NP:], y2, d, b_out.reshape(1, D0),
              batch_p.reshape(NP, 1))
    return out
